# baseline (speedup 1.0000x reference)
import functools

import jax
import jax.numpy as jnp
from jax import lax
from jax.experimental import pallas as pl
from jax.experimental.pallas import tpu as pltpu
from jax.experimental.pallas import tpu_sc as plsc

_NC = 2
_NS = 16
_NW = _NC * _NS
_BN = 128
_C = 8
_ROWS = 2048
_ED = 168

_N_SC = 8192


def _ssp_sc_kernel(n_sc: int):
  bpw = n_sc // _BN // _NW
  mesh = plsc.VectorSubcoreMesh(core_axis_name="c", subcore_axis_name="s")

  @functools.partial(
      pl.kernel,
      out_type=jax.ShapeDtypeStruct((_ED, n_sc), jnp.float32),
      mesh=mesh,
      scratch_types=[
          pltpu.VMEM((256, _BN), jnp.float32),
          pltpu.VMEM((256, _BN), jnp.float32),
          pltpu.VMEM((2, _ED, _BN), jnp.float32),
          pltpu.SemaphoreType.DMA,
          pltpu.SemaphoreType.DMA,
          pltpu.SemaphoreType.DMA,
      ],
      compiler_params=pltpu.CompilerParams(needs_layout_passes=False),
  )
  def ssp(pt_hbm, emb_hbm, buf0, buf1, ebuf, sem0, sem1, sem_out):
    wid = lax.axis_index("s") * _NC + lax.axis_index("c")
    bufs = (buf0, buf1)
    sems = (sem0, sem1)
    col_base = wid * bpw * _BN

    def issue_in(t, slot):
      t = jnp.minimum(t, bpw * _C - 1)
      k = t // _C
      c = t % _C
      pltpu.async_copy(
          pt_hbm.at[pl.ds(c * 256, 256), pl.ds(col_base + k * _BN, _BN)],
          bufs[slot], sems[slot])

    def wait_in(slot):
      pltpu.make_async_copy(
          pt_hbm.at[pl.ds(0, 256), pl.ds(0, _BN)], bufs[slot],
          sems[slot]).wait()

    def wait_out():
      pltpu.make_async_copy(
          ebuf.at[0], emb_hbm.at[:, pl.ds(0, _BN)], sem_out).wait()

    issue_in(0, 0)
    issue_in(1, 1)

    def block_body(k, carry):
      km2 = k % 2

      @pl.when(k >= 2)
      def _():
        wait_out()

      for c in range(_C):
        t = k * _C + c
        wait_in(c % 2)
        buf = bufs[c % 2]

        def gj_body(gj, carry2):
          accs = []
          for v in range(_BN // 16):
            col = pl.ds(v * 16, 16)
            acc = None
            for dy in range(4):
              for dx in range(4):
                val = buf[(gj // 4 * 4 + dy) * 16 + (gj % 4) * 4 + dx, col]
                acc = val if acc is None else jnp.maximum(acc, val)
            accs.append(acc)
          for v in range(_BN // 16):
            ebuf[km2, 40 + c * 16 + gj, pl.ds(v * 16, 16)] = accs[v]
          return carry2

        lax.fori_loop(0, 16, gj_body, 0, unroll=False)

        for i in range(2):
          for j in range(2):
            for v in range(_BN // 16):
              col = pl.ds(v * 16, 16)
              acc = None
              for di in range(2):
                for dj in range(2):
                  val = ebuf[km2, 40 + c * 16 + (2 * i + di) * 4 +
                             (2 * j + dj), col]
                  acc = val if acc is None else jnp.maximum(acc, val)
              ebuf[km2, 8 + c * 4 + i * 2 + j, col] = acc
        for v in range(_BN // 16):
          col = pl.ds(v * 16, 16)
          acc = None
          for q in range(4):
            val = ebuf[km2, 8 + c * 4 + q, col]
            acc = val if acc is None else jnp.maximum(acc, val)
          ebuf[km2, c, col] = acc

        issue_in(t + 2, c % 2)

      pltpu.async_copy(
          ebuf.at[km2], emb_hbm.at[:, pl.ds(col_base + k * _BN, _BN)],
          sem_out)
      return carry

    lax.fori_loop(0, bpw, block_body, 0, unroll=False)

    wait_in(0)
    wait_in(1)
    wait_out()
    wait_out()

  return ssp


def _dense_tail(emb, t):
  s = jnp.sum(emb * emb, axis=0, keepdims=True)
  emb = emb / jnp.maximum(jnp.sqrt(s), 1e-12)
  return emb


def _router_math(emb, keys, t):
  s = jnp.sum(emb * emb, axis=0, keepdims=True)
  emb = emb / jnp.maximum(jnp.sqrt(s), 1e-12)
  logits = lax.dot_general(
      keys, emb, (((1,), (0,)), ((), ())),
      preferred_element_type=jnp.float32)
  m = jnp.max(logits, axis=0, keepdims=True)
  e = jnp.exp(logits - m)
  w = e / jnp.sum(e, axis=0, keepdims=True)
  wf = jnp.where(w > t, w, 0.0)
  return wf / (jnp.sum(wf, axis=0, keepdims=True) + 1e-8)


def _fused_body(thr_ref, x_ref, keys_ref, out_ref):
  x = x_ref[...]
  bn = x.shape[1]
  r = x.reshape(8, 4, 4, 4, 4, bn)
  p4 = r.max(axis=4).max(axis=2)
  r2 = p4.reshape(8, 2, 2, 2, 2, bn)
  p2 = r2.max(axis=4).max(axis=2)
  p1 = p2.max(axis=2).max(axis=1)
  emb = jnp.concatenate(
      [p1, p2.reshape(32, bn), p4.reshape(128, bn)], axis=0)
  out_ref[...] = _router_math(emb, keys_ref[...], thr_ref[0])


def _dense_body(thr_ref, emb_ref, keys_ref, prev_ref, out_ref):
  del prev_ref
  out_ref[...] = _router_math(emb_ref[...], keys_ref[...], thr_ref[0])


def kernel(patch, keys, threshold):
  n = patch.shape[0]
  pt = jnp.transpose(patch, (1, 2, 3, 0)).reshape(_ROWS, n)
  thr = jnp.reshape(threshold, (1,))
  n_tc = n - _N_SC

  emb_sc = _ssp_sc_kernel(_N_SC)(pt)

  bn_f = 512
  off = _N_SC // bn_f
  out1 = pl.pallas_call(
      _fused_body,
      grid=(n_tc // bn_f,),
      in_specs=[
          pl.BlockSpec(memory_space=pltpu.SMEM),
          pl.BlockSpec((_ROWS, bn_f), lambda i: (0, i + off)),
          pl.BlockSpec((64, _ED), lambda i: (0, 0)),
      ],
      out_specs=pl.BlockSpec((64, bn_f), lambda i: (0, i + off)),
      out_shape=jax.ShapeDtypeStruct((64, n), jnp.float32),
  )(thr, pt, keys)

  bn_d = 2048
  out_t = pl.pallas_call(
      _dense_body,
      grid=(_N_SC // bn_d,),
      in_specs=[
          pl.BlockSpec(memory_space=pltpu.SMEM),
          pl.BlockSpec((_ED, bn_d), lambda i: (0, i)),
          pl.BlockSpec((64, _ED), lambda i: (0, 0)),
          pl.BlockSpec(memory_space=pl.ANY),
      ],
      out_specs=pl.BlockSpec((64, bn_d), lambda i: (0, i)),
      out_shape=jax.ShapeDtypeStruct((64, n), jnp.float32),
      input_output_aliases={3: 0},
  )(thr, emb_sc, keys, out1)

  return jnp.transpose(out_t)


# speedup vs baseline: 1.0368x; 1.0368x over previous
import functools

import jax
import jax.numpy as jnp
from jax import lax
from jax.experimental import pallas as pl
from jax.experimental.pallas import tpu as pltpu
from jax.experimental.pallas import tpu_sc as plsc

_NC = 2
_NS = 16
_NW = _NC * _NS
_BN = 128
_C = 8
_ROWS = 2048
_ED = 168

_N_SC = 8192


def _ssp_sc_kernel(n_sc: int):
  bpw = n_sc // _BN // _NW
  mesh = plsc.VectorSubcoreMesh(core_axis_name="c", subcore_axis_name="s")

  @functools.partial(
      pl.kernel,
      out_type=jax.ShapeDtypeStruct((_ED, n_sc), jnp.float32),
      mesh=mesh,
      scratch_types=[
          pltpu.VMEM((256, _BN), jnp.float32),
          pltpu.VMEM((256, _BN), jnp.float32),
          pltpu.VMEM((2, _ED, _BN), jnp.float32),
          pltpu.SemaphoreType.DMA,
          pltpu.SemaphoreType.DMA,
          pltpu.SemaphoreType.DMA,
      ],
      compiler_params=pltpu.CompilerParams(needs_layout_passes=False),
  )
  def ssp(pt_hbm, emb_hbm, buf0, buf1, ebuf, sem0, sem1, sem_out):
    wid = lax.axis_index("s") * _NC + lax.axis_index("c")
    bufs = (buf0, buf1)
    sems = (sem0, sem1)
    col_base = wid * bpw * _BN

    def issue_in(t, slot):
      t = jnp.minimum(t, bpw * _C - 1)
      k = t // _C
      c = t % _C
      pltpu.async_copy(
          pt_hbm.at[pl.ds(c * 256, 256), pl.ds(col_base + k * _BN, _BN)],
          bufs[slot], sems[slot])

    def wait_in(slot):
      pltpu.make_async_copy(
          pt_hbm.at[pl.ds(0, 256), pl.ds(0, _BN)], bufs[slot],
          sems[slot]).wait()

    def wait_out():
      pltpu.make_async_copy(
          ebuf.at[0], emb_hbm.at[:, pl.ds(0, _BN)], sem_out).wait()

    issue_in(0, 0)
    issue_in(1, 1)

    def block_body(k, carry):
      km2 = k % 2

      @pl.when(k >= 2)
      def _():
        wait_out()

      for c in range(_C):
        t = k * _C + c
        wait_in(c % 2)
        buf = bufs[c % 2]

        def gj_body(gj, carry2):
          accs = []
          for v in range(_BN // 16):
            col = pl.ds(v * 16, 16)
            acc = None
            for dy in range(4):
              for dx in range(4):
                val = buf[(gj // 4 * 4 + dy) * 16 + (gj % 4) * 4 + dx, col]
                acc = val if acc is None else jnp.maximum(acc, val)
            accs.append(acc)
          for v in range(_BN // 16):
            ebuf[km2, 40 + c * 16 + gj, pl.ds(v * 16, 16)] = accs[v]
          return carry2

        lax.fori_loop(0, 16, gj_body, 0, unroll=False)

        for i in range(2):
          for j in range(2):
            for v in range(_BN // 16):
              col = pl.ds(v * 16, 16)
              acc = None
              for di in range(2):
                for dj in range(2):
                  val = ebuf[km2, 40 + c * 16 + (2 * i + di) * 4 +
                             (2 * j + dj), col]
                  acc = val if acc is None else jnp.maximum(acc, val)
              ebuf[km2, 8 + c * 4 + i * 2 + j, col] = acc
        for v in range(_BN // 16):
          col = pl.ds(v * 16, 16)
          acc = None
          for q in range(4):
            val = ebuf[km2, 8 + c * 4 + q, col]
            acc = val if acc is None else jnp.maximum(acc, val)
          ebuf[km2, c, col] = acc

        issue_in(t + 2, c % 2)

      pltpu.async_copy(
          ebuf.at[km2], emb_hbm.at[:, pl.ds(col_base + k * _BN, _BN)],
          sem_out)
      return carry

    lax.fori_loop(0, bpw, block_body, 0, unroll=False)

    wait_in(0)
    wait_in(1)
    wait_out()
    wait_out()

  return ssp


def _dense_tail(emb, t):
  s = jnp.sum(emb * emb, axis=0, keepdims=True)
  emb = emb / jnp.maximum(jnp.sqrt(s), 1e-12)
  return emb


def _router_math(emb, keys, t):
  s = jnp.sum(emb * emb, axis=0, keepdims=True)
  emb = emb / jnp.maximum(jnp.sqrt(s), 1e-12)
  logits = lax.dot_general(
      keys, emb, (((1,), (0,)), ((), ())),
      preferred_element_type=jnp.float32)
  m = jnp.max(logits, axis=0, keepdims=True)
  e = jnp.exp(logits - m)
  w = e / jnp.sum(e, axis=0, keepdims=True)
  wf = jnp.where(w > t, w, 0.0)
  return wf / (jnp.sum(wf, axis=0, keepdims=True) + 1e-8)


def _fused_body(thr_ref, x_ref, keys_ref, out_ref):
  x = x_ref[...]
  bn = x.shape[1]
  r = x.reshape(8, 4, 4, 4, 4, bn)
  p4 = r.max(axis=4).max(axis=2)
  r2 = p4.reshape(8, 2, 2, 2, 2, bn)
  p2 = r2.max(axis=4).max(axis=2)
  p1 = p2.max(axis=2).max(axis=1)
  emb = jnp.concatenate(
      [p1, p2.reshape(32, bn), p4.reshape(128, bn)], axis=0)
  out_ref[...] = _router_math(emb, keys_ref[...], thr_ref[0])


def _dense_body(thr_ref, emb_ref, keys_ref, prev_ref, out_ref):
  del prev_ref
  out_ref[...] = _router_math(emb_ref[...], keys_ref[...], thr_ref[0])


def kernel(patch, keys, threshold):
  n = patch.shape[0]
  pt = jnp.transpose(patch, (1, 2, 3, 0)).reshape(_ROWS, n)
  thr = jnp.reshape(threshold, (1,))
  n_tc = n - _N_SC

  emb_sc = _ssp_sc_kernel(_N_SC)(pt)

  bn_f = 1024
  off = _N_SC // bn_f
  out1 = pl.pallas_call(
      _fused_body,
      grid=(n_tc // bn_f,),
      in_specs=[
          pl.BlockSpec(memory_space=pltpu.SMEM),
          pl.BlockSpec((_ROWS, bn_f), lambda i: (0, i + off)),
          pl.BlockSpec((64, _ED), lambda i: (0, 0)),
      ],
      out_specs=pl.BlockSpec((64, bn_f), lambda i: (0, i + off)),
      out_shape=jax.ShapeDtypeStruct((64, n), jnp.float32),
  )(thr, pt, keys)

  bn_d = 4096
  out_t = pl.pallas_call(
      _dense_body,
      grid=(_N_SC // bn_d,),
      in_specs=[
          pl.BlockSpec(memory_space=pltpu.SMEM),
          pl.BlockSpec((_ED, bn_d), lambda i: (0, i)),
          pl.BlockSpec((64, _ED), lambda i: (0, 0)),
          pl.BlockSpec(memory_space=pl.ANY),
      ],
      out_specs=pl.BlockSpec((64, bn_d), lambda i: (0, i)),
      out_shape=jax.ShapeDtypeStruct((64, n), jnp.float32),
      input_output_aliases={3: 0},
  )(thr, emb_sc, keys, out1)

  return jnp.transpose(out_t)


# speedup vs baseline: 1.0372x; 1.0004x over previous
import functools

import jax
import jax.numpy as jnp
from jax import lax
from jax.experimental import pallas as pl
from jax.experimental.pallas import tpu as pltpu
from jax.experimental.pallas import tpu_sc as plsc

_NC = 2
_NS = 16
_NW = _NC * _NS
_BN = 128
_C = 8
_ROWS = 2048
_ED = 168

_N_SC = 8192


def _ssp_sc_kernel(n_sc: int):
  bpw = n_sc // _BN // _NW
  mesh = plsc.VectorSubcoreMesh(core_axis_name="c", subcore_axis_name="s")

  @functools.partial(
      pl.kernel,
      out_type=jax.ShapeDtypeStruct((_ED, n_sc), jnp.float32),
      mesh=mesh,
      scratch_types=[
          pltpu.VMEM((256, _BN), jnp.float32),
          pltpu.VMEM((256, _BN), jnp.float32),
          pltpu.VMEM((2, _ED, _BN), jnp.float32),
          pltpu.SemaphoreType.DMA,
          pltpu.SemaphoreType.DMA,
          pltpu.SemaphoreType.DMA,
      ],
      compiler_params=pltpu.CompilerParams(needs_layout_passes=False),
  )
  def ssp(pt_hbm, emb_hbm, buf0, buf1, ebuf, sem0, sem1, sem_out):
    wid = lax.axis_index("s") * _NC + lax.axis_index("c")
    bufs = (buf0, buf1)
    sems = (sem0, sem1)
    col_base = wid * bpw * _BN

    def issue_in(t, slot):
      t = jnp.minimum(t, bpw * _C - 1)
      k = t // _C
      c = t % _C
      pltpu.async_copy(
          pt_hbm.at[pl.ds(c * 256, 256), pl.ds(col_base + k * _BN, _BN)],
          bufs[slot], sems[slot])

    def wait_in(slot):
      pltpu.make_async_copy(
          pt_hbm.at[pl.ds(0, 256), pl.ds(0, _BN)], bufs[slot],
          sems[slot]).wait()

    def wait_out():
      pltpu.make_async_copy(
          ebuf.at[0], emb_hbm.at[:, pl.ds(0, _BN)], sem_out).wait()

    issue_in(0, 0)
    issue_in(1, 1)

    def block_body(k, carry):
      km2 = k % 2

      @pl.when(k >= 2)
      def _():
        wait_out()

      for c in range(_C):
        t = k * _C + c
        wait_in(c % 2)
        buf = bufs[c % 2]

        def gj_body(gj, carry2):
          accs = []
          for v in range(_BN // 16):
            col = pl.ds(v * 16, 16)
            acc = None
            for dy in range(4):
              for dx in range(4):
                val = buf[(gj // 4 * 4 + dy) * 16 + (gj % 4) * 4 + dx, col]
                acc = val if acc is None else jnp.maximum(acc, val)
            accs.append(acc)
          for v in range(_BN // 16):
            ebuf[km2, 40 + c * 16 + gj, pl.ds(v * 16, 16)] = accs[v]
          return carry2

        lax.fori_loop(0, 16, gj_body, 0, unroll=False)

        for i in range(2):
          for j in range(2):
            for v in range(_BN // 16):
              col = pl.ds(v * 16, 16)
              acc = None
              for di in range(2):
                for dj in range(2):
                  val = ebuf[km2, 40 + c * 16 + (2 * i + di) * 4 +
                             (2 * j + dj), col]
                  acc = val if acc is None else jnp.maximum(acc, val)
              ebuf[km2, 8 + c * 4 + i * 2 + j, col] = acc
        for v in range(_BN // 16):
          col = pl.ds(v * 16, 16)
          acc = None
          for q in range(4):
            val = ebuf[km2, 8 + c * 4 + q, col]
            acc = val if acc is None else jnp.maximum(acc, val)
          ebuf[km2, c, col] = acc

        issue_in(t + 2, c % 2)

      pltpu.async_copy(
          ebuf.at[km2], emb_hbm.at[:, pl.ds(col_base + k * _BN, _BN)],
          sem_out)
      return carry

    lax.fori_loop(0, bpw, block_body, 0, unroll=False)

    wait_in(0)
    wait_in(1)
    wait_out()
    wait_out()

  return ssp


def _router_math(emb, keys, t):
  s = jnp.sum(emb * emb, axis=0, keepdims=True)
  emb = emb / jnp.maximum(jnp.sqrt(s), 1e-12)
  logits = lax.dot_general(
      keys, emb, (((1,), (0,)), ((), ())),
      preferred_element_type=jnp.float32)
  m = jnp.max(logits, axis=0, keepdims=True)
  e = jnp.exp(logits - m)
  w = e / jnp.sum(e, axis=0, keepdims=True)
  wf = jnp.where(w > t, w, 0.0)
  return wf / (jnp.sum(wf, axis=0, keepdims=True) + 1e-8)


def _fused_body(thr_ref, x_ref, keys_ref, out_ref):
  x = x_ref[...]
  bn = x.shape[1]
  r = x.reshape(8, 4, 4, 4, 4, bn)
  p4 = r.max(axis=4).max(axis=2)
  r2 = p4.reshape(8, 2, 2, 2, 2, bn)
  p2 = r2.max(axis=4).max(axis=2)
  p1 = p2.max(axis=2).max(axis=1)
  emb = jnp.concatenate(
      [p1, p2.reshape(32, bn), p4.reshape(128, bn)], axis=0)
  out_ref[...] = _router_math(emb, keys_ref[...], thr_ref[0])


def _dense_body(thr_ref, emb_ref, keys_ref, prev_ref, out_ref):
  del prev_ref
  out_ref[...] = _router_math(emb_ref[...], keys_ref[...], thr_ref[0])


def kernel(patch, keys, threshold):
  n = patch.shape[0]
  pt = jnp.transpose(patch, (1, 2, 3, 0)).reshape(_ROWS, n)
  thr = jnp.reshape(threshold, (1,))
  n_tc = n - _N_SC

  emb_sc = _ssp_sc_kernel(_N_SC)(pt)

  bn_f = 1024
  off = _N_SC // bn_f
  out1 = pl.pallas_call(
      _fused_body,
      grid=(n_tc // bn_f,),
      in_specs=[
          pl.BlockSpec(memory_space=pltpu.SMEM),
          pl.BlockSpec((_ROWS, bn_f), lambda i: (0, i + off)),
          pl.BlockSpec((64, _ED), lambda i: (0, 0)),
      ],
      out_specs=pl.BlockSpec((64, bn_f), lambda i: (0, i + off)),
      out_shape=jax.ShapeDtypeStruct((64, n), jnp.float32),
  )(thr, pt, keys)

  bn_d = 4096
  out_t = pl.pallas_call(
      _dense_body,
      grid=(_N_SC // bn_d,),
      in_specs=[
          pl.BlockSpec(memory_space=pltpu.SMEM),
          pl.BlockSpec((_ED, bn_d), lambda i: (0, i)),
          pl.BlockSpec((64, _ED), lambda i: (0, 0)),
          pl.BlockSpec(memory_space=pl.ANY),
      ],
      out_specs=pl.BlockSpec((64, bn_d), lambda i: (0, i)),
      out_shape=jax.ShapeDtypeStruct((64, n), jnp.float32),
      input_output_aliases={3: 0},
  )(thr, emb_sc, keys, out1)

  return jnp.transpose(out_t)
